# Initial kernel scaffold; baseline (speedup 1.0000x reference)
import jax
import jax.numpy as jnp
from jax.experimental import pallas as pl


def kernel(feature, edge_index, W, a, bias):
    raise NotImplementedError("write your pallas kernel here")


# speedup vs baseline: 15.9372x; 15.9372x over previous
import functools

import jax
import jax.numpy as jnp
from jax import lax
from jax.experimental import pallas as pl
from jax.experimental.pallas import tpu as pltpu
from jax.experimental.pallas import tpu_sc as plsc

N = 10000
E = 320000
D = 128
NPAD = 10240
NTILES = 32
EPT = E // NTILES
K = 80
NCHUNK = EPT // K
RPT = NPAD // 16
ZCOPIES = RPT // K


_MBLK = 400


def _wh_body(f_ref, wt_ref, a1_ref, a2_ref, wh_ref, s_ref, t_ref):
    wh = jnp.dot(f_ref[...], wt_ref[...], preferred_element_type=jnp.float32)
    wh_ref[...] = wh
    s_ref[...] = jnp.dot(wh, a1_ref[...].T, preferred_element_type=jnp.float32)
    t_ref[...] = jnp.dot(wh, a2_ref[...].T, preferred_element_type=jnp.float32)


def _wh_call(feature, wt, a1, a2):
    return pl.pallas_call(
        _wh_body,
        grid=(N // _MBLK,),
        in_specs=[
            pl.BlockSpec((_MBLK, D), lambda i: (i, 0)),
            pl.BlockSpec((D, D), lambda i: (0, 0)),
            pl.BlockSpec((1, D), lambda i: (0, 0)),
            pl.BlockSpec((1, D), lambda i: (0, 0)),
        ],
        out_specs=[
            pl.BlockSpec((_MBLK, D), lambda i: (i, 0)),
            pl.BlockSpec((_MBLK, 1), lambda i: (i, 0)),
            pl.BlockSpec((_MBLK, 1), lambda i: (i, 0)),
        ],
        out_shape=[
            jax.ShapeDtypeStruct((N, D), jnp.float32),
            jax.ShapeDtypeStruct((N, 1), jnp.float32),
            jax.ShapeDtypeStruct((N, 1), jnp.float32),
        ],
    )(feature, wt, a1, a2)


_sc_mesh = plsc.VectorSubcoreMesh(core_axis_name="c", subcore_axis_name="s")


@functools.partial(
    pl.kernel,
    out_type=[
        jax.ShapeDtypeStruct((2, NPAD, D), jnp.float32),
        jax.ShapeDtypeStruct((2, NPAD), jnp.float32),
    ],
    mesh=_sc_mesh,
    compiler_params=pltpu.CompilerParams(needs_layout_passes=False),
    scratch_types=[
        pltpu.VMEM_SHARED((NPAD, D), jnp.float32),
        pltpu.VMEM_SHARED((NPAD,), jnp.float32),
        pltpu.VMEM((N,), jnp.float32),
        pltpu.VMEM((N,), jnp.float32),
        pltpu.VMEM((K,), jnp.int32),
        pltpu.VMEM((K,), jnp.int32),
        pltpu.VMEM((K, D), jnp.float32),
        pltpu.VMEM((K,), jnp.float32),
        pltpu.VMEM((RPT,), jnp.float32),
        pltpu.SemaphoreType.DMA,
    ],
)
def _sc_edges(wh_hbm, s_hbm, t_hbm, src_hbm, dst_hbm,
              numer_out, denom_out,
              numer_sh, denom_sh, s_v, t_v, src_v, dst_v, rows_v, p_v,
              z1d, sem):
    cid = lax.axis_index("c")
    sid = lax.axis_index("s")
    row0 = sid * RPT
    _ZV = jnp.zeros((16,), jnp.float32)

    def _zero_rows(i, _):
        for j in range(D // 16):
            rows_v[i, pl.ds(j * 16, 16)] = _ZV
        return _

    lax.fori_loop(0, K, _zero_rows, None)

    def _zero_z1(i, _):
        z1d[pl.ds(i * 16, 16)] = _ZV
        return _

    lax.fori_loop(0, RPT // 16, _zero_z1, None)

    def _fill_numer(c, _):
        pltpu.sync_copy(rows_v, numer_sh.at[pl.ds(row0 + c * K, K)])
        return _

    lax.fori_loop(0, ZCOPIES, _fill_numer, None)
    pltpu.sync_copy(z1d, denom_sh.at[pl.ds(row0, RPT)])

    pltpu.sync_copy(s_hbm, s_v)
    pltpu.sync_copy(t_hbm, t_v)

    plsc.subcore_barrier()

    estart = (cid * 16 + sid) * EPT

    def _chunk(g, _):
        base = estart + g * K
        pltpu.sync_copy(src_hbm.at[pl.ds(base, K)], src_v)
        pltpu.sync_copy(dst_hbm.at[pl.ds(base, K)], dst_v)
        gather = pltpu.async_copy(wh_hbm.at[src_v], rows_v, sem)

        def _weights(j, _):
            si = src_v[pl.ds(j * 16, 16)]
            di = dst_v[pl.ds(j * 16, 16)]
            e = plsc.load_gather(s_v, [si]) + plsc.load_gather(t_v, [di])
            e = jnp.where(e >= 0.0, e, 0.2 * e)
            p_v[pl.ds(j * 16, 16)] = jnp.exp(e)
            return _

        lax.fori_loop(0, K // 16, _weights, None)
        gather.wait()

        def _scale(i, _):
            pvec = p_v[pl.ds(i * 16, 16)]
            for r in range(16):
                p = pvec[r]
                row = i * 16 + r
                for j in range(D // 16):
                    rows_v[row, pl.ds(j * 16, 16)] = (
                        rows_v[row, pl.ds(j * 16, 16)] * p)
            return _

        lax.fori_loop(0, K // 16, _scale, None)

        pltpu.sync_copy(rows_v, numer_sh.at[dst_v], add=True)
        pltpu.sync_copy(p_v, denom_sh.at[dst_v], add=True)
        return _

    lax.fori_loop(0, NCHUNK, _chunk, None)

    plsc.subcore_barrier()

    def _dump(c, _):
        r = row0 + c * K
        pltpu.sync_copy(numer_sh.at[pl.ds(r, K)], rows_v)
        pltpu.sync_copy(rows_v, numer_out.at[cid, pl.ds(r, K)])
        return _

    lax.fori_loop(0, ZCOPIES, _dump, None)
    pltpu.sync_copy(denom_sh.at[pl.ds(row0, RPT)], z1d)
    pltpu.sync_copy(z1d, denom_out.at[cid, pl.ds(row0, RPT)])


def _final_body(n_ref, d_ref, b_ref, o_ref):
    n = n_ref[...]
    d = d_ref[...]
    num = n[0] + n[1]
    den = d[0] + d[1]
    h = jnp.where(den > 0.0, num / jnp.where(den > 0.0, den, 1.0), 0.0)
    x = h + b_ref[...]
    o_ref[...] = jnp.where(x > 0.0, x, jnp.exp(jnp.minimum(x, 0.0)) - 1.0)


def _final_call(numer, denom3, bias):
    return pl.pallas_call(
        _final_body,
        grid=(N // _MBLK,),
        in_specs=[
            pl.BlockSpec((2, _MBLK, D), lambda i: (0, i, 0)),
            pl.BlockSpec((2, _MBLK, 1), lambda i: (0, i, 0)),
            pl.BlockSpec((1, D), lambda i: (0, 0)),
        ],
        out_specs=pl.BlockSpec((_MBLK, D), lambda i: (i, 0)),
        out_shape=jax.ShapeDtypeStruct((N, D), jnp.float32),
    )(numer, denom3, bias)


def kernel(feature, edge_index, W, a, bias):
    wt = W.T
    a1 = a[:, :D]
    a2 = a[:, D:]
    wh, s2, t2 = _wh_call(feature, wt, a1, a2)
    s = s2.reshape(N)
    t = t2.reshape(N)
    src = edge_index[0]
    dst = edge_index[1]
    numer, denom = _sc_edges(wh, s, t, src, dst)
    return _final_call(numer, denom.reshape(2, NPAD, 1), bias)
